# Initial kernel scaffold; baseline (speedup 1.0000x reference)
import functools

import jax
import jax.numpy as jnp
from jax import lax
from jax.experimental import pallas as pl
from jax.experimental.pallas import tpu as pltpu
from jax.experimental.pallas import tpu_sc as plsc

_K = 128


@functools.lru_cache(maxsize=None)
def _make_sc_gather(n_items, n_users, B, E, d, n_negs):
    info = plsc.get_sparse_core_info()
    NC, NS = info.num_cores, info.num_subcores
    NW = NC * NS
    EW = E // NW
    NCH = EW // _K
    BW = B // NW
    NQ = BW // _K
    NEG = B * n_negs
    NEGW = NEG // NW
    NNCH = NEGW // _K
    RPT = B // NS

    mesh = plsc.VectorSubcoreMesh(core_axis_name="c", subcore_axis_name="s")

    @functools.partial(
        pl.kernel,
        out_type=(
            jax.ShapeDtypeStruct((NC, B, d), jnp.float32),
            jax.ShapeDtypeStruct((B, d), jnp.float32),
            jax.ShapeDtypeStruct((B, d), jnp.float32),
            jax.ShapeDtypeStruct((NEG, d), jnp.float32),
            jax.ShapeDtypeStruct((B,), jnp.float32),
            jax.ShapeDtypeStruct((B,), jnp.float32),
            jax.ShapeDtypeStruct((NEG,), jnp.float32),
        ),
        mesh=mesh,
        scratch_types=[
            pltpu.VMEM((NCH, _K), jnp.int32),
            pltpu.VMEM((NCH, _K), jnp.int32),
            pltpu.VMEM((_K, d), jnp.float32),
            pltpu.VMEM((NNCH, _K), jnp.int32),
            pltpu.VMEM((_K,), jnp.float32),
            pltpu.VMEM_SHARED((B, d), jnp.float32),
            pltpu.SemaphoreType.DMA,
        ],
    )
    def sc_kernel(p_hbm, q_hbm, bu_hbm, bi_hbm, iu2, us2, i2, u2, ineg2, z_hbm,
                  psum_out, q_out, pself_out, qneg_out, bu_out, bi_out, bineg_out,
                  iu_v, us_v, rows_v, idx_v, bvec_v, psum_sh, sem):
        cid = lax.axis_index("c")
        sid = lax.axis_index("s")
        wid = cid * NS + sid

        pltpu.sync_copy(z_hbm, rows_v)
        for j in range(RPT // _K):
            pltpu.sync_copy(rows_v, psum_sh.at[pl.ds(sid * RPT + j * _K, _K)])
        plsc.subcore_barrier()

        pltpu.sync_copy(iu2.at[pl.ds(wid * NCH, NCH)], iu_v)
        pltpu.sync_copy(us2.at[pl.ds(wid * NCH, NCH)], us_v)

        @pl.loop(0, NCH)
        def _edges(j):
            pltpu.async_copy(p_hbm.at[iu_v.at[j]], rows_v, sem).wait()
            pltpu.sync_copy(rows_v, psum_sh.at[us_v.at[j]], add=True)

        plsc.subcore_barrier()

        pltpu.sync_copy(psum_sh.at[pl.ds(sid * RPT, RPT)],
                        psum_out.at[cid, pl.ds(sid * RPT, RPT)])

        pltpu.sync_copy(i2.at[pl.ds(wid * NQ, NQ)], idx_v.at[pl.ds(0, NQ)])
        for j in range(NQ):
            base = wid * BW + j * _K
            pltpu.async_copy(q_hbm.at[idx_v.at[j]], rows_v, sem).wait()
            pltpu.sync_copy(rows_v, q_out.at[pl.ds(base, _K)])
            pltpu.async_copy(p_hbm.at[idx_v.at[j]], rows_v, sem).wait()
            pltpu.sync_copy(rows_v, pself_out.at[pl.ds(base, _K)])
            pltpu.async_copy(bi_hbm.at[idx_v.at[j]], bvec_v, sem).wait()
            pltpu.sync_copy(bvec_v, bi_out.at[pl.ds(base, _K)])

        pltpu.sync_copy(u2.at[pl.ds(wid * NQ, NQ)], idx_v.at[pl.ds(0, NQ)])
        for j in range(NQ):
            base = wid * BW + j * _K
            pltpu.async_copy(bu_hbm.at[idx_v.at[j]], bvec_v, sem).wait()
            pltpu.sync_copy(bvec_v, bu_out.at[pl.ds(base, _K)])

        pltpu.sync_copy(ineg2.at[pl.ds(wid * NNCH, NNCH)], idx_v)
        for j in range(NNCH):
            base = wid * NEGW + j * _K
            pltpu.async_copy(q_hbm.at[idx_v.at[j]], rows_v, sem).wait()
            pltpu.sync_copy(rows_v, qneg_out.at[pl.ds(base, _K)])
            pltpu.async_copy(bi_hbm.at[idx_v.at[j]], bvec_v, sem).wait()
            pltpu.sync_copy(bvec_v, bineg_out.at[pl.ds(base, _K)])

    return sc_kernel, NC, NW, NCH, NNCH


def _tc_score(psum, q, pself, qnegT, bu, bi, binegT, B, d, n_negs, NC):
    BLK = 2048
    grid = (B // BLK,)

    def body(psum_ref, q_ref, pself_ref, qneg_ref, bu_ref, bi_ref, bineg_ref,
             r_ref, rnegt_ref):
        p_sum = psum_ref[0]
        for c in range(1, NC):
            p_sum = p_sum + psum_ref[c]
        p_ctx = p_sum - pself_ref[...]
        pq = jnp.sum(p_ctx * q_ref[...], axis=1)
        r_ref[...] = bu_ref[...] + bi_ref[...] + pq
        pqn = jnp.sum(p_ctx[None] * qneg_ref[...], axis=2)
        rnegt_ref[...] = bu_ref[...][None, :] + bineg_ref[...] + pqn

    return pl.pallas_call(
        body,
        grid=grid,
        in_specs=[
            pl.BlockSpec((NC, BLK, d), lambda i: (0, i, 0)),
            pl.BlockSpec((BLK, d), lambda i: (i, 0)),
            pl.BlockSpec((BLK, d), lambda i: (i, 0)),
            pl.BlockSpec((n_negs, BLK, d), lambda i: (0, i, 0)),
            pl.BlockSpec((BLK,), lambda i: (i,)),
            pl.BlockSpec((BLK,), lambda i: (i,)),
            pl.BlockSpec((n_negs, BLK), lambda i: (0, i)),
        ],
        out_specs=[
            pl.BlockSpec((BLK,), lambda i: (i,)),
            pl.BlockSpec((n_negs, BLK), lambda i: (0, i)),
        ],
        out_shape=[
            jax.ShapeDtypeStruct((B,), jnp.float32),
            jax.ShapeDtypeStruct((n_negs, B), jnp.float32),
        ],
    )(psum, q, pself, qnegT, bu, bi, binegT)


def kernel(P_table, Q_table, b_u, b_i, I, U, I_neg, I_U, U_idx):
    B = I.shape[0]
    n_negs = I_neg.shape[1]
    E = I_U.shape[0]
    n_items, d = P_table.shape
    n_users = b_u.shape[0]

    sc_kernel, NC, NW, NCH, NNCH = _make_sc_gather(n_items, n_users, B, E, d, n_negs)

    iu2 = I_U.astype(jnp.int32).reshape(NW * NCH, _K)
    us2 = U_idx.astype(jnp.int32).reshape(NW * NCH, _K)
    i2 = I.astype(jnp.int32).reshape(B // _K, _K)
    u2 = U.astype(jnp.int32).reshape(B // _K, _K)
    ineg2 = I_neg.astype(jnp.int32).T.reshape(NW * NNCH, _K)
    zeros = jnp.zeros((_K, d), jnp.float32)

    psum, q, pself, qneg, bu, bi, bineg = sc_kernel(
        P_table, Q_table, b_u, b_i, iu2, us2, i2, u2, ineg2, zeros)

    qnegT = qneg.reshape(n_negs, B, d)
    binegT = bineg.reshape(n_negs, B)
    r, rnegT = _tc_score(psum, q, pself, qnegT, bu, bi, binegT, B, d, n_negs, NC)
    return (r, rnegT.T)


# speedup vs baseline: 9.9475x; 9.9475x over previous
import functools

import jax
import jax.numpy as jnp
from jax import lax
from jax.experimental import pallas as pl
from jax.experimental.pallas import tpu as pltpu
from jax.experimental.pallas import tpu_sc as plsc

_K = 128


@functools.lru_cache(maxsize=None)
def _make_sc_gather(n_items, n_users, B, E, d, n_negs):
    info = plsc.get_sparse_core_info()
    NC, NS = info.num_cores, info.num_subcores
    NW = NC * NS
    EW = E // NW
    NCH = EW // _K
    BW = B // NW
    NQ = BW // _K
    NEG = B * n_negs
    NEGW = NEG // NW
    NNCH = NEGW // _K
    RPT = B // NS

    mesh = plsc.VectorSubcoreMesh(core_axis_name="c", subcore_axis_name="s")

    @functools.partial(
        pl.kernel,
        out_type=(
            jax.ShapeDtypeStruct((NC, B, d), jnp.float32),
            jax.ShapeDtypeStruct((B, d), jnp.float32),
            jax.ShapeDtypeStruct((B, d), jnp.float32),
            jax.ShapeDtypeStruct((NEG, d), jnp.float32),
            jax.ShapeDtypeStruct((B,), jnp.float32),
            jax.ShapeDtypeStruct((B,), jnp.float32),
            jax.ShapeDtypeStruct((NEG,), jnp.float32),
        ),
        mesh=mesh,
        compiler_params=pltpu.CompilerParams(use_tc_tiling_on_sc=False),
        scratch_types=[
            pltpu.VMEM((NCH, _K), jnp.int32),
            pltpu.VMEM((NCH, _K), jnp.int32),
            pltpu.VMEM((_K, d), jnp.float32),
            pltpu.VMEM((NNCH, _K), jnp.int32),
            pltpu.VMEM((_K,), jnp.float32),
            pltpu.VMEM_SHARED((B, d), jnp.float32),
            pltpu.SemaphoreType.DMA,
        ],
    )
    def sc_kernel(p_hbm, q_hbm, bu_hbm, bi_hbm, iu2, us2, i2, u2, ineg2, z_hbm,
                  psum_out, q_out, pself_out, qneg_out, bu_out, bi_out, bineg_out,
                  iu_v, us_v, rows_v, idx_v, bvec_v, psum_sh, sem):
        cid = lax.axis_index("c")
        sid = lax.axis_index("s")
        wid = cid * NS + sid

        pltpu.sync_copy(z_hbm, rows_v)
        for j in range(RPT // _K):
            pltpu.sync_copy(rows_v, psum_sh.at[pl.ds(sid * RPT + j * _K, _K)])
        plsc.subcore_barrier()

        pltpu.sync_copy(iu2.at[pl.ds(wid * NCH, NCH)], iu_v)
        pltpu.sync_copy(us2.at[pl.ds(wid * NCH, NCH)], us_v)

        @pl.loop(0, NCH)
        def _edges(j):
            pltpu.async_copy(p_hbm.at[iu_v.at[j]], rows_v, sem).wait()
            pltpu.sync_copy(rows_v, psum_sh.at[us_v.at[j]], add=True)

        plsc.subcore_barrier()

        pltpu.sync_copy(psum_sh.at[pl.ds(sid * RPT, RPT)],
                        psum_out.at[cid, pl.ds(sid * RPT, RPT)])

        pltpu.sync_copy(i2.at[pl.ds(wid * NQ, NQ)], idx_v.at[pl.ds(0, NQ)])
        for j in range(NQ):
            base = wid * BW + j * _K
            pltpu.async_copy(q_hbm.at[idx_v.at[j]], rows_v, sem).wait()
            pltpu.sync_copy(rows_v, q_out.at[pl.ds(base, _K)])
            pltpu.async_copy(p_hbm.at[idx_v.at[j]], rows_v, sem).wait()
            pltpu.sync_copy(rows_v, pself_out.at[pl.ds(base, _K)])
            pltpu.async_copy(bi_hbm.at[idx_v.at[j]], bvec_v, sem).wait()
            pltpu.sync_copy(bvec_v, bi_out.at[pl.ds(base, _K)])

        pltpu.sync_copy(u2.at[pl.ds(wid * NQ, NQ)], idx_v.at[pl.ds(0, NQ)])
        for j in range(NQ):
            base = wid * BW + j * _K
            pltpu.async_copy(bu_hbm.at[idx_v.at[j]], bvec_v, sem).wait()
            pltpu.sync_copy(bvec_v, bu_out.at[pl.ds(base, _K)])

        pltpu.sync_copy(ineg2.at[pl.ds(wid * NNCH, NNCH)], idx_v)
        for j in range(NNCH):
            base = wid * NEGW + j * _K
            pltpu.async_copy(q_hbm.at[idx_v.at[j]], rows_v, sem).wait()
            pltpu.sync_copy(rows_v, qneg_out.at[pl.ds(base, _K)])
            pltpu.async_copy(bi_hbm.at[idx_v.at[j]], bvec_v, sem).wait()
            pltpu.sync_copy(bvec_v, bineg_out.at[pl.ds(base, _K)])

    return sc_kernel, NC, NW, NCH, NNCH


def _tc_score(psum, q, pself, qnegT, bu, bi, binegT, B, d, n_negs, NC):
    BLK = 2048
    grid = (B // BLK,)

    def body(psum_ref, q_ref, pself_ref, qneg_ref, bu_ref, bi_ref, bineg_ref,
             r_ref, rnegt_ref):
        p_sum = psum_ref[0]
        for c in range(1, NC):
            p_sum = p_sum + psum_ref[c]
        p_ctx = p_sum - pself_ref[...]
        pq = jnp.sum(p_ctx * q_ref[...], axis=1)
        r_ref[...] = bu_ref[...] + bi_ref[...] + pq
        pqn = jnp.sum(p_ctx[None] * qneg_ref[...], axis=2)
        rnegt_ref[...] = bu_ref[...][None, :] + bineg_ref[...] + pqn

    return pl.pallas_call(
        body,
        grid=grid,
        in_specs=[
            pl.BlockSpec((NC, BLK, d), lambda i: (0, i, 0)),
            pl.BlockSpec((BLK, d), lambda i: (i, 0)),
            pl.BlockSpec((BLK, d), lambda i: (i, 0)),
            pl.BlockSpec((n_negs, BLK, d), lambda i: (0, i, 0)),
            pl.BlockSpec((BLK,), lambda i: (i,)),
            pl.BlockSpec((BLK,), lambda i: (i,)),
            pl.BlockSpec((n_negs, BLK), lambda i: (0, i)),
        ],
        out_specs=[
            pl.BlockSpec((BLK,), lambda i: (i,)),
            pl.BlockSpec((n_negs, BLK), lambda i: (0, i)),
        ],
        out_shape=[
            jax.ShapeDtypeStruct((B,), jnp.float32),
            jax.ShapeDtypeStruct((n_negs, B), jnp.float32),
        ],
    )(psum, q, pself, qnegT, bu, bi, binegT)


def kernel(P_table, Q_table, b_u, b_i, I, U, I_neg, I_U, U_idx):
    B = I.shape[0]
    n_negs = I_neg.shape[1]
    E = I_U.shape[0]
    n_items, d = P_table.shape
    n_users = b_u.shape[0]

    sc_kernel, NC, NW, NCH, NNCH = _make_sc_gather(n_items, n_users, B, E, d, n_negs)

    iu2 = I_U.astype(jnp.int32).reshape(NW * NCH, _K)
    us2 = U_idx.astype(jnp.int32).reshape(NW * NCH, _K)
    i2 = I.astype(jnp.int32).reshape(B // _K, _K)
    u2 = U.astype(jnp.int32).reshape(B // _K, _K)
    ineg2 = I_neg.astype(jnp.int32).T.reshape(NW * NNCH, _K)
    zeros = jnp.zeros((_K, d), jnp.float32)

    psum, q, pself, qneg, bu, bi, bineg = sc_kernel(
        P_table, Q_table, b_u, b_i, iu2, us2, i2, u2, ineg2, zeros)

    qnegT = qneg.reshape(n_negs, B, d)
    binegT = bineg.reshape(n_negs, B)
    r, rnegT = _tc_score(psum, q, pself, qnegT, bu, bi, binegT, B, d, n_negs, NC)
    return (r, rnegT.T)


# speedup vs baseline: 10.7497x; 1.0806x over previous
import functools

import jax
import jax.numpy as jnp
from jax import lax
from jax.experimental import pallas as pl
from jax.experimental.pallas import tpu as pltpu
from jax.experimental.pallas import tpu_sc as plsc

_K = 128
_G = 10


@functools.lru_cache(maxsize=None)
def _make_sc_gather(n_items, n_users, B, E, d, n_negs):
    info = plsc.get_sparse_core_info()
    NC, NS = info.num_cores, info.num_subcores
    NW = NC * NS
    EW = E // NW
    NCH = EW // _K
    BW = B // NW
    NQ = BW // _K
    NEG = B * n_negs
    NEGW = NEG // NW
    NNCH = NEGW // _K
    RPT = B // NS

    mesh = plsc.VectorSubcoreMesh(core_axis_name="c", subcore_axis_name="s")

    @functools.partial(
        pl.kernel,
        out_type=(
            jax.ShapeDtypeStruct((NC, B, d), jnp.float32),
            jax.ShapeDtypeStruct((B, d), jnp.float32),
            jax.ShapeDtypeStruct((B, d), jnp.float32),
            jax.ShapeDtypeStruct((NEG, d), jnp.float32),
            jax.ShapeDtypeStruct((B,), jnp.float32),
            jax.ShapeDtypeStruct((B,), jnp.float32),
            jax.ShapeDtypeStruct((NEG,), jnp.float32),
        ),
        mesh=mesh,
        compiler_params=pltpu.CompilerParams(use_tc_tiling_on_sc=False),
        scratch_types=[
            pltpu.VMEM((_G, _K), jnp.int32),
            pltpu.VMEM((_G, _K), jnp.int32),
            pltpu.VMEM((_K, d), jnp.float32),
            pltpu.VMEM((_K, d), jnp.float32),
            pltpu.VMEM((2 * (B // _K // NW) + (NEG // _K // NW), _K), jnp.int32),
            pltpu.VMEM((_K,), jnp.float32),
            pltpu.VMEM((_K,), jnp.float32),
            pltpu.VMEM_SHARED((B, d), jnp.float32),
            pltpu.SemaphoreType.DMA,
            pltpu.SemaphoreType.DMA,
            pltpu.SemaphoreType.DMA,
            pltpu.SemaphoreType.DMA,
            pltpu.SemaphoreType.DMA,
        ],
    )
    def sc_kernel(p_hbm, q_hbm, bu_hbm, bi_hbm, iu2, us2, i2, u2, ineg2, z_hbm,
                  psum_out, q_out, pself_out, qneg_out, bu_out, bi_out, bineg_out,
                  iu_g, us_g, rows_a, rows_b, idx_v, bva, bvb, psum_sh,
                  gsem, rsem_a, rsem_b, bsem_a, bsem_b):
        rsems = (rsem_a, rsem_b)
        bsems = (bsem_a, bsem_b)
        cid = lax.axis_index("c")
        sid = lax.axis_index("s")
        wid = cid * NS + sid
        rbufs = (rows_a, rows_b)

        pltpu.sync_copy(z_hbm, rows_a)
        for j in range(RPT // _K):
            pltpu.sync_copy(rows_a, psum_sh.at[pl.ds(sid * RPT + j * _K, _K)])

        pltpu.sync_copy(i2.at[pl.ds(wid * NQ, NQ)], idx_v.at[pl.ds(0, NQ)])
        pltpu.sync_copy(u2.at[pl.ds(wid * NQ, NQ)], idx_v.at[pl.ds(NQ, NQ)])
        pltpu.sync_copy(ineg2.at[pl.ds(wid * NNCH, NNCH)],
                        idx_v.at[pl.ds(2 * NQ, NNCH)])

        tasks = []
        for j in range(NQ):
            base = wid * BW + j * _K
            tasks.append((q_hbm, j, q_out.at[pl.ds(base, _K)],
                          bi_hbm, j, bi_out.at[pl.ds(base, _K)]))
            tasks.append((p_hbm, j, pself_out.at[pl.ds(base, _K)],
                          bu_hbm, NQ + j, bu_out.at[pl.ds(base, _K)]))
        for j in range(NNCH):
            base = wid * NEGW + j * _K
            tasks.append((q_hbm, 2 * NQ + j, qneg_out.at[pl.ds(base, _K)],
                          bi_hbm, 2 * NQ + j, bineg_out.at[pl.ds(base, _K)]))

        def fire(t):
            tab, ji, _, btab, bji, _ = tasks[t]
            descs = [pltpu.async_copy(tab.at[idx_v.at[ji]], rbufs[t % 2],
                                      rsems[t % 2])]
            if btab is not None:
                descs.append(pltpu.async_copy(btab.at[idx_v.at[bji]],
                                              bbufs[t % 2], bsems[t % 2]))
            return descs

        bbufs = (bva, bvb)
        pend = fire(0)
        for t in range(len(tasks)):
            nxt = fire(t + 1) if t + 1 < len(tasks) else []
            for dsc in pend:
                dsc.wait()
            pend = nxt
            _, _, out, btab, _, bout = tasks[t]
            pltpu.sync_copy(rbufs[t % 2], out)
            if btab is not None:
                pltpu.sync_copy(bbufs[t % 2], bout)

        plsc.subcore_barrier()

        @pl.loop(0, NCH, step=_G)
        def _edges(j):
            pltpu.sync_copy(iu2.at[pl.ds(wid * NCH + j, _G)], iu_g)
            pltpu.sync_copy(us2.at[pl.ds(wid * NCH + j, _G)], us_g)
            pltpu.async_copy(p_hbm.at[iu_g.at[0]], rows_a, gsem)
            for k in range(_G):
                cur = rbufs[k % 2]
                pltpu.make_async_copy(p_hbm.at[iu_g.at[k]], cur, gsem).wait()
                if k + 1 < _G:
                    pltpu.async_copy(p_hbm.at[iu_g.at[k + 1]],
                                     rbufs[(k + 1) % 2], gsem)
                pltpu.sync_copy(cur, psum_sh.at[us_g.at[k]], add=True)

        plsc.subcore_barrier()

        pltpu.sync_copy(psum_sh.at[pl.ds(sid * RPT, RPT)],
                        psum_out.at[cid, pl.ds(sid * RPT, RPT)])

    return sc_kernel, NC, NW, NCH, NNCH, NQ


def _tc_score(psum, q, pself, qneg3, bu, bi, bineg2, B, d, n_negs, NC):
    BLK = 2048
    grid = (B // BLK,)

    def body(psum_ref, q_ref, pself_ref, qneg_ref, bu_ref, bi_ref, bineg_ref,
             r_ref, rneg_ref):
        p_sum = psum_ref[0]
        for c in range(1, NC):
            p_sum = p_sum + psum_ref[c]
        p_ctx = p_sum - pself_ref[...]
        pq = jnp.sum(p_ctx * q_ref[...], axis=1)
        r_ref[...] = bu_ref[...] + bi_ref[...] + pq
        pqn = jnp.sum(p_ctx[:, None, :] * qneg_ref[...], axis=2)
        rneg_ref[...] = bu_ref[...][:, None] + bineg_ref[...] + pqn

    return pl.pallas_call(
        body,
        grid=grid,
        in_specs=[
            pl.BlockSpec((NC, BLK, d), lambda i: (0, i, 0)),
            pl.BlockSpec((BLK, d), lambda i: (i, 0)),
            pl.BlockSpec((BLK, d), lambda i: (i, 0)),
            pl.BlockSpec((BLK, n_negs, d), lambda i: (i, 0, 0)),
            pl.BlockSpec((BLK,), lambda i: (i,)),
            pl.BlockSpec((BLK,), lambda i: (i,)),
            pl.BlockSpec((BLK, n_negs), lambda i: (i, 0)),
        ],
        out_specs=[
            pl.BlockSpec((BLK,), lambda i: (i,)),
            pl.BlockSpec((BLK, n_negs), lambda i: (i, 0)),
        ],
        out_shape=[
            jax.ShapeDtypeStruct((B,), jnp.float32),
            jax.ShapeDtypeStruct((B, n_negs), jnp.float32),
        ],
    )(psum, q, pself, qneg3, bu, bi, bineg2)


def kernel(P_table, Q_table, b_u, b_i, I, U, I_neg, I_U, U_idx):
    B = I.shape[0]
    n_negs = I_neg.shape[1]
    E = I_U.shape[0]
    n_items, d = P_table.shape
    n_users = b_u.shape[0]

    sc_kernel, NC, NW, NCH, NNCH, NQ = _make_sc_gather(
        n_items, n_users, B, E, d, n_negs)

    iu2 = I_U.astype(jnp.int32).reshape(NW * NCH, _K)
    us2 = U_idx.astype(jnp.int32).reshape(NW * NCH, _K)
    i2 = I.astype(jnp.int32).reshape(B // _K, _K)
    u2 = U.astype(jnp.int32).reshape(B // _K, _K)
    ineg2 = I_neg.astype(jnp.int32).reshape(NW * NNCH, _K)
    zeros = jnp.zeros((_K, d), jnp.float32)

    psum, q, pself, qneg, bu, bi, bineg = sc_kernel(
        P_table, Q_table, b_u, b_i, iu2, us2, i2, u2, ineg2, zeros)

    qneg3 = qneg.reshape(B, n_negs, d)
    bineg2 = bineg.reshape(B, n_negs)
    r, rneg = _tc_score(psum, q, pself, qneg3, bu, bi, bineg2, B, d, n_negs, NC)
    return (r, rneg)


# speedup vs baseline: 10.7582x; 1.0008x over previous
import functools

import jax
import jax.numpy as jnp
from jax import lax
from jax.experimental import pallas as pl
from jax.experimental.pallas import tpu as pltpu
from jax.experimental.pallas import tpu_sc as plsc

_K = 128
_G = 10


@functools.lru_cache(maxsize=None)
def _make_sc_gather(n_items, n_users, B, E, d, n_negs):
    info = plsc.get_sparse_core_info()
    NC, NS = info.num_cores, info.num_subcores
    NW = NC * NS
    EW = E // NW
    NCH = EW // _K
    BW = B // NW
    NQ = BW // _K
    NEG = B * n_negs
    NEGW = NEG // NW
    NNCH = NEGW // _K
    RPT = B // NS

    mesh = plsc.VectorSubcoreMesh(core_axis_name="c", subcore_axis_name="s")

    @functools.partial(
        pl.kernel,
        out_type=(
            jax.ShapeDtypeStruct((NC, B, d), jnp.float32),
            jax.ShapeDtypeStruct((B, d), jnp.float32),
            jax.ShapeDtypeStruct((B, d), jnp.float32),
            jax.ShapeDtypeStruct((NEG, d), jnp.float32),
            jax.ShapeDtypeStruct((B,), jnp.float32),
            jax.ShapeDtypeStruct((B,), jnp.float32),
            jax.ShapeDtypeStruct((NEG,), jnp.float32),
        ),
        mesh=mesh,
        compiler_params=pltpu.CompilerParams(use_tc_tiling_on_sc=False),
        scratch_types=[
            pltpu.VMEM((_G * _K,), jnp.int32),
            pltpu.VMEM((_G, _K), jnp.int32),
            pltpu.VMEM((_K, d), jnp.float32),
            pltpu.VMEM((_K, d), jnp.float32),
            pltpu.VMEM(((2 * (B // _K // NW) + (NEG // _K // NW)) * _K,),
                       jnp.int32),
            pltpu.VMEM((_K,), jnp.float32),
            pltpu.VMEM((_K,), jnp.float32),
            pltpu.VMEM_SHARED((B, d), jnp.float32),
            pltpu.SemaphoreType.DMA,
            pltpu.SemaphoreType.DMA,
            pltpu.SemaphoreType.DMA,
            pltpu.SemaphoreType.DMA,
            pltpu.SemaphoreType.DMA,
        ],
    )
    def sc_kernel(p_hbm, q_hbm, bu_hbm, bi_hbm, iu1, us1, i1, u1, ineg1, z_hbm,
                  psum_out, q_out, pself_out, qneg_out, bu_out, bi_out, bineg_out,
                  iu_g, us_g, rows_a, rows_b, idx_v, bva, bvb, psum_sh,
                  gsem, rsem_a, rsem_b, bsem_a, bsem_b):

        rsems = (rsem_a, rsem_b)
        bsems = (bsem_a, bsem_b)
        cid = lax.axis_index("c")
        sid = lax.axis_index("s")
        wid = cid * NS + sid
        rbufs = (rows_a, rows_b)

        pltpu.sync_copy(z_hbm, rows_a)
        for j in range(RPT // _K):
            pltpu.sync_copy(rows_a, psum_sh.at[pl.ds(sid * RPT + j * _K, _K)])

        pltpu.sync_copy(i1.at[pl.ds(wid * BW, BW)], idx_v.at[pl.ds(0, BW)])
        pltpu.sync_copy(u1.at[pl.ds(wid * BW, BW)], idx_v.at[pl.ds(BW, BW)])
        pltpu.sync_copy(ineg1.at[pl.ds(wid * NEGW, NEGW)],
                        idx_v.at[pl.ds(2 * BW, NEGW)])

        tasks = []
        for j in range(NQ):
            base = wid * BW + j * _K
            tasks.append((q_hbm, j, q_out.at[pl.ds(base, _K)],
                          bi_hbm, j, bi_out.at[pl.ds(base, _K)]))
            tasks.append((p_hbm, j, pself_out.at[pl.ds(base, _K)],
                          bu_hbm, NQ + j, bu_out.at[pl.ds(base, _K)]))
        for j in range(NNCH):
            base = wid * NEGW + j * _K
            tasks.append((q_hbm, 2 * NQ + j, qneg_out.at[pl.ds(base, _K)],
                          bi_hbm, 2 * NQ + j, bineg_out.at[pl.ds(base, _K)]))

        def fire(t):
            tab, ji, _, btab, bji, _ = tasks[t]
            descs = [pltpu.async_copy(tab.at[idx_v.at[pl.ds(ji * _K, _K)]],
                                      rbufs[t % 2], rsems[t % 2])]
            if btab is not None:
                descs.append(pltpu.async_copy(
                    btab.at[idx_v.at[pl.ds(bji * _K, _K)]],
                    bbufs[t % 2], bsems[t % 2]))
            return descs

        bbufs = (bva, bvb)
        pend = fire(0)
        for t in range(len(tasks)):
            nxt = fire(t + 1) if t + 1 < len(tasks) else []
            for dsc in pend:
                dsc.wait()
            pend = nxt
            _, _, out, btab, _, bout = tasks[t]
            pltpu.sync_copy(rbufs[t % 2], out)
            if btab is not None:
                pltpu.sync_copy(bbufs[t % 2], bout)

        plsc.subcore_barrier()

        @pl.loop(0, NCH, step=_G)
        def _edges(j):
            ebase = wid * EW + j * _K
            pltpu.sync_copy(iu1.at[pl.ds(ebase, _G * _K)], iu_g)
            sdescs = [pltpu.async_copy(us1.at[pl.ds(ebase + k * _K, _K)],
                                       us_g.at[k], rsem_a) for k in range(_G)]
            for dsc in sdescs:
                dsc.wait()
            pltpu.async_copy(p_hbm.at[iu_g.at[pl.ds(0, _K)]], rows_a, gsem)
            for k in range(_G):
                cur = rbufs[k % 2]
                pltpu.make_async_copy(p_hbm.at[iu_g.at[pl.ds(k * _K, _K)]],
                                      cur, gsem).wait()
                if k + 1 < _G:
                    pltpu.async_copy(
                        p_hbm.at[iu_g.at[pl.ds((k + 1) * _K, _K)]],
                        rbufs[(k + 1) % 2], gsem)
                pltpu.sync_copy(cur, psum_sh.at[us_g.at[k]], add=True)

        plsc.subcore_barrier()

        pltpu.sync_copy(psum_sh.at[pl.ds(sid * RPT, RPT)],
                        psum_out.at[cid, pl.ds(sid * RPT, RPT)])

    return sc_kernel, NC, NW, NCH, NNCH, NQ


def _tc_score(psum, q, pself, qneg3, bu, bi, bineg2, B, d, n_negs, NC):
    BLK = 2048
    grid = (B // BLK,)

    def body(psum_ref, q_ref, pself_ref, qneg_ref, bu_ref, bi_ref, bineg_ref,
             r_ref, rneg_ref):
        p_sum = psum_ref[0]
        for c in range(1, NC):
            p_sum = p_sum + psum_ref[c]
        p_ctx = p_sum - pself_ref[...]
        pq = jnp.sum(p_ctx * q_ref[...], axis=1)
        r_ref[...] = bu_ref[...] + bi_ref[...] + pq
        pqn = jnp.sum(p_ctx[:, None, :] * qneg_ref[...], axis=2)
        rneg_ref[...] = bu_ref[...][:, None] + bineg_ref[...] + pqn

    return pl.pallas_call(
        body,
        grid=grid,
        in_specs=[
            pl.BlockSpec((NC, BLK, d), lambda i: (0, i, 0)),
            pl.BlockSpec((BLK, d), lambda i: (i, 0)),
            pl.BlockSpec((BLK, d), lambda i: (i, 0)),
            pl.BlockSpec((BLK, n_negs, d), lambda i: (i, 0, 0)),
            pl.BlockSpec((BLK,), lambda i: (i,)),
            pl.BlockSpec((BLK,), lambda i: (i,)),
            pl.BlockSpec((BLK, n_negs), lambda i: (i, 0)),
        ],
        out_specs=[
            pl.BlockSpec((BLK,), lambda i: (i,)),
            pl.BlockSpec((BLK, n_negs), lambda i: (i, 0)),
        ],
        out_shape=[
            jax.ShapeDtypeStruct((B,), jnp.float32),
            jax.ShapeDtypeStruct((B, n_negs), jnp.float32),
        ],
    )(psum, q, pself, qneg3, bu, bi, bineg2)


def kernel(P_table, Q_table, b_u, b_i, I, U, I_neg, I_U, U_idx):
    B = I.shape[0]
    n_negs = I_neg.shape[1]
    E = I_U.shape[0]
    n_items, d = P_table.shape
    n_users = b_u.shape[0]

    sc_kernel, NC, NW, NCH, NNCH, NQ = _make_sc_gather(
        n_items, n_users, B, E, d, n_negs)

    iu1 = I_U.astype(jnp.int32)
    us1 = U_idx.astype(jnp.int32)
    i1 = I.astype(jnp.int32)
    u1 = U.astype(jnp.int32)
    ineg1 = I_neg.astype(jnp.int32).reshape(-1)
    zeros = jnp.zeros((_K, d), jnp.float32)

    psum, q, pself, qneg, bu, bi, bineg = sc_kernel(
        P_table, Q_table, b_u, b_i, iu1, us1, i1, u1, ineg1, zeros)

    qneg3 = qneg.reshape(B, n_negs, d)
    bineg2 = bineg.reshape(B, n_negs)
    r, rneg = _tc_score(psum, q, pself, qneg3, bu, bi, bineg2, B, d, n_negs, NC)
    return (r, rneg)


# speedup vs baseline: 12.0798x; 1.1228x over previous
import functools

import jax
import jax.numpy as jnp
from jax import lax
from jax.experimental import pallas as pl
from jax.experimental.pallas import tpu as pltpu
from jax.experimental.pallas import tpu_sc as plsc

_K = 128
_G = 10


@functools.lru_cache(maxsize=None)
def _make_sc_gather(n_items, n_users, B, E, d, n_negs):
    info = plsc.get_sparse_core_info()
    NC, NS = info.num_cores, info.num_subcores
    NW = NC * NS
    EW = E // NW
    NCH = EW // _K
    BW = B // NW
    NQ = BW // _K
    NEG = B * n_negs
    NEGW = NEG // NW
    NNCH = NEGW // _K
    RPT = B // NS

    mesh = plsc.VectorSubcoreMesh(core_axis_name="c", subcore_axis_name="s")

    @functools.partial(
        pl.kernel,
        out_type=(
            jax.ShapeDtypeStruct((NC, B, 128), jnp.float32),
            jax.ShapeDtypeStruct((B, 128), jnp.float32),
            jax.ShapeDtypeStruct((B, 128), jnp.float32),
            jax.ShapeDtypeStruct((NEG, 128), jnp.float32),
            jax.ShapeDtypeStruct((B,), jnp.float32),
            jax.ShapeDtypeStruct((B,), jnp.float32),
            jax.ShapeDtypeStruct((NEG,), jnp.float32),
        ),
        mesh=mesh,
        compiler_params=pltpu.CompilerParams(use_tc_tiling_on_sc=False),
        scratch_types=[
            pltpu.VMEM((_G * _K,), jnp.int32),
            pltpu.VMEM((_G, _K), jnp.int32),
            pltpu.VMEM((_K, d), jnp.float32),
            pltpu.VMEM((_K, d), jnp.float32),
            pltpu.VMEM(((2 * (B // _K // NW) + (NEG // _K // NW)) * _K,),
                       jnp.int32),
            pltpu.VMEM((_K,), jnp.float32),
            pltpu.VMEM((_K,), jnp.float32),
            pltpu.VMEM_SHARED((B, d), jnp.float32),
            pltpu.SemaphoreType.DMA,
            pltpu.SemaphoreType.DMA,
            pltpu.SemaphoreType.DMA,
            pltpu.SemaphoreType.DMA,
            pltpu.SemaphoreType.DMA,
        ],
    )
    def sc_kernel(p_hbm, q_hbm, bu_hbm, bi_hbm, iu1, us1, i1, u1, ineg1, z_hbm,
                  psum_out, q_out, pself_out, qneg_out, bu_out, bi_out, bineg_out,
                  iu_g, us_g, rows_a, rows_b, idx_v, bva, bvb, psum_sh,
                  gsem, rsem_a, rsem_b, bsem_a, bsem_b):

        rsems = (rsem_a, rsem_b)
        bsems = (bsem_a, bsem_b)
        cid = lax.axis_index("c")
        sid = lax.axis_index("s")
        wid = cid * NS + sid
        rbufs = (rows_a, rows_b)

        pltpu.sync_copy(z_hbm, rows_a)
        for j in range(RPT // _K):
            pltpu.sync_copy(rows_a, psum_sh.at[pl.ds(sid * RPT + j * _K, _K)])

        pltpu.sync_copy(i1.at[pl.ds(wid * BW, BW)], idx_v.at[pl.ds(0, BW)])
        pltpu.sync_copy(u1.at[pl.ds(wid * BW, BW)], idx_v.at[pl.ds(BW, BW)])
        pltpu.sync_copy(ineg1.at[pl.ds(wid * NEGW, NEGW)],
                        idx_v.at[pl.ds(2 * BW, NEGW)])

        tasks = []
        for j in range(NQ):
            base = wid * BW + j * _K
            tasks.append((q_hbm, j, q_out.at[pl.ds(base, _K), pl.ds(0, d)],
                          bi_hbm, j, bi_out.at[pl.ds(base, _K)]))
            tasks.append((p_hbm, j,
                          pself_out.at[pl.ds(base, _K), pl.ds(0, d)],
                          bu_hbm, NQ + j, bu_out.at[pl.ds(base, _K)]))
        for j in range(NNCH):
            base = wid * NEGW + j * _K
            tasks.append((q_hbm, 2 * NQ + j,
                          qneg_out.at[pl.ds(base, _K), pl.ds(0, d)],
                          bi_hbm, 2 * NQ + j, bineg_out.at[pl.ds(base, _K)]))

        def fire(t):
            tab, ji, _, btab, bji, _ = tasks[t]
            descs = [pltpu.async_copy(tab.at[idx_v.at[pl.ds(ji * _K, _K)]],
                                      rbufs[t % 2], rsems[t % 2])]
            if btab is not None:
                descs.append(pltpu.async_copy(
                    btab.at[idx_v.at[pl.ds(bji * _K, _K)]],
                    bbufs[t % 2], bsems[t % 2]))
            return descs

        bbufs = (bva, bvb)
        pend = fire(0)
        for t in range(len(tasks)):
            nxt = fire(t + 1) if t + 1 < len(tasks) else []
            for dsc in pend:
                dsc.wait()
            pend = nxt
            _, _, out, btab, _, bout = tasks[t]
            pltpu.sync_copy(rbufs[t % 2], out)
            if btab is not None:
                pltpu.sync_copy(bbufs[t % 2], bout)

        plsc.subcore_barrier()

        @pl.loop(0, NCH, step=_G)
        def _edges(j):
            ebase = wid * EW + j * _K
            pltpu.sync_copy(iu1.at[pl.ds(ebase, _G * _K)], iu_g)
            sdescs = [pltpu.async_copy(us1.at[pl.ds(ebase + k * _K, _K)],
                                       us_g.at[k], rsem_a) for k in range(_G)]
            for dsc in sdescs:
                dsc.wait()
            pltpu.async_copy(p_hbm.at[iu_g.at[pl.ds(0, _K)]], rows_a, gsem)
            for k in range(_G):
                cur = rbufs[k % 2]
                pltpu.make_async_copy(p_hbm.at[iu_g.at[pl.ds(k * _K, _K)]],
                                      cur, gsem).wait()
                if k + 1 < _G:
                    pltpu.async_copy(
                        p_hbm.at[iu_g.at[pl.ds((k + 1) * _K, _K)]],
                        rbufs[(k + 1) % 2], gsem)
                pltpu.sync_copy(cur, psum_sh.at[us_g.at[k]], add=True)

        plsc.subcore_barrier()

        pltpu.sync_copy(psum_sh.at[pl.ds(sid * RPT, RPT)],
                        psum_out.at[cid, pl.ds(sid * RPT, RPT), pl.ds(0, d)])

    return sc_kernel, NC, NW, NCH, NNCH, NQ


def _tc_score(psum, q, pself, qneg3, bu, bi, bineg2, B, d, n_negs, NC):
    BLK = 2048
    grid = (B // BLK,)

    def body(psum_ref, q_ref, pself_ref, qneg_ref, bu_ref, bi_ref, bineg_ref,
             r_ref, rneg_ref):
        p_sum = psum_ref[0, :, :d]
        for c in range(1, NC):
            p_sum = p_sum + psum_ref[c, :, :d]
        p_ctx = p_sum - pself_ref[:, :d]
        pq = jnp.sum(p_ctx * q_ref[:, :d], axis=1)
        r_ref[...] = bu_ref[...] + bi_ref[...] + pq
        pqn = jnp.sum(p_ctx[:, None, :] * qneg_ref[:, :, :d], axis=2)
        rneg_ref[...] = bu_ref[...][:, None] + bineg_ref[...] + pqn

    return pl.pallas_call(
        body,
        grid=grid,
        in_specs=[
            pl.BlockSpec((NC, BLK, 128), lambda i: (0, i, 0)),
            pl.BlockSpec((BLK, 128), lambda i: (i, 0)),
            pl.BlockSpec((BLK, 128), lambda i: (i, 0)),
            pl.BlockSpec((BLK, n_negs, 128), lambda i: (i, 0, 0)),
            pl.BlockSpec((BLK,), lambda i: (i,)),
            pl.BlockSpec((BLK,), lambda i: (i,)),
            pl.BlockSpec((BLK, n_negs), lambda i: (i, 0)),
        ],
        out_specs=[
            pl.BlockSpec((BLK,), lambda i: (i,)),
            pl.BlockSpec((BLK, n_negs), lambda i: (i, 0)),
        ],
        out_shape=[
            jax.ShapeDtypeStruct((B,), jnp.float32),
            jax.ShapeDtypeStruct((B, n_negs), jnp.float32),
        ],
    )(psum, q, pself, qneg3, bu, bi, bineg2)


def kernel(P_table, Q_table, b_u, b_i, I, U, I_neg, I_U, U_idx):
    B = I.shape[0]
    n_negs = I_neg.shape[1]
    E = I_U.shape[0]
    n_items, d = P_table.shape
    n_users = b_u.shape[0]

    sc_kernel, NC, NW, NCH, NNCH, NQ = _make_sc_gather(
        n_items, n_users, B, E, d, n_negs)

    iu1 = I_U.astype(jnp.int32)
    us1 = U_idx.astype(jnp.int32)
    i1 = I.astype(jnp.int32)
    u1 = U.astype(jnp.int32)
    ineg1 = I_neg.astype(jnp.int32).reshape(-1)
    zeros = jnp.zeros((_K, d), jnp.float32)

    psum, q, pself, qneg, bu, bi, bineg = sc_kernel(
        P_table, Q_table, b_u, b_i, iu1, us1, i1, u1, ineg1, zeros)

    qneg3 = qneg.reshape(B, n_negs, 128)
    bineg2 = bineg.reshape(B, n_negs)
    r, rneg = _tc_score(psum, q, pself, qneg3, bu, bi, bineg2, B, d, n_negs, NC)
    return (r, rneg)


# speedup vs baseline: 14.1815x; 1.1740x over previous
import functools

import jax
import jax.numpy as jnp
from jax import lax
from jax.experimental import pallas as pl
from jax.experimental.pallas import tpu as pltpu
from jax.experimental.pallas import tpu_sc as plsc

_K = 128
_G = 10


@functools.lru_cache(maxsize=None)
def _make_sc_gather(n_items, n_users, B, E, d, n_negs):
    info = plsc.get_sparse_core_info()
    NC, NS = info.num_cores, info.num_subcores
    NW = NC * NS
    EW = E // NW
    NCH = EW // _K
    BW = B // NW
    NQ = BW // _K
    NEG = B * n_negs
    NEGW = NEG // NW
    NNCH = NEGW // _K
    RPT = B // NS

    mesh = plsc.VectorSubcoreMesh(core_axis_name="c", subcore_axis_name="s")

    @functools.partial(
        pl.kernel,
        out_type=(
            jax.ShapeDtypeStruct((NC, B, 128), jnp.float32),
            jax.ShapeDtypeStruct((B, 128), jnp.float32),
            jax.ShapeDtypeStruct((B, 128), jnp.float32),
            jax.ShapeDtypeStruct((NEG, 128), jnp.float32),
            jax.ShapeDtypeStruct((B,), jnp.float32),
            jax.ShapeDtypeStruct((B,), jnp.float32),
            jax.ShapeDtypeStruct((NEG,), jnp.float32),
        ),
        mesh=mesh,
        compiler_params=pltpu.CompilerParams(use_tc_tiling_on_sc=False),
        scratch_types=[
            pltpu.VMEM((_G * _K,), jnp.int32),
            pltpu.VMEM((_G, _K), jnp.int32),
            pltpu.VMEM((_K, d), jnp.float32),
            pltpu.VMEM((_K, d), jnp.float32),
            pltpu.VMEM((_K, d), jnp.float32),
            pltpu.VMEM(((2 * (B // _K // NW) + (NEG // _K // NW)) * _K,),
                       jnp.int32),
            pltpu.VMEM((_K,), jnp.float32),
            pltpu.VMEM((_K,), jnp.float32),
            pltpu.VMEM_SHARED((B, d), jnp.float32),
            pltpu.SemaphoreType.DMA,
            pltpu.SemaphoreType.DMA,
            pltpu.SemaphoreType.DMA,
            pltpu.SemaphoreType.DMA,
            pltpu.SemaphoreType.DMA,
            pltpu.SemaphoreType.DMA,
            pltpu.SemaphoreType.DMA,
        ],
    )
    def sc_kernel(p_hbm, q_hbm, bu_hbm, bi_hbm, iu1, us1, i1, u1, ineg1, z_hbm,
                  psum_out, q_out, pself_out, qneg_out, bu_out, bi_out, bineg_out,
                  iu_g, us_g, rows_a, rows_b, rows_c, idx_v, bva, bvb, psum_sh,
                  gsem_a, gsem_b, gsem_c, rsem_a, rsem_b, bsem_a, bsem_b):
        gbufs = (rows_a, rows_b, rows_c)
        gsems = (gsem_a, gsem_b, gsem_c)

        rsems = (rsem_a, rsem_b)
        bsems = (bsem_a, bsem_b)
        cid = lax.axis_index("c")
        sid = lax.axis_index("s")
        wid = cid * NS + sid
        rbufs = (rows_a, rows_b)

        pltpu.sync_copy(z_hbm, rows_a)
        for j in range(RPT // _K):
            pltpu.sync_copy(rows_a, psum_sh.at[pl.ds(sid * RPT + j * _K, _K)])

        pltpu.sync_copy(i1.at[pl.ds(wid * BW, BW)], idx_v.at[pl.ds(0, BW)])
        pltpu.sync_copy(u1.at[pl.ds(wid * BW, BW)], idx_v.at[pl.ds(BW, BW)])
        pltpu.sync_copy(ineg1.at[pl.ds(wid * NEGW, NEGW)],
                        idx_v.at[pl.ds(2 * BW, NEGW)])

        tasks = []
        for j in range(NQ):
            base = wid * BW + j * _K
            tasks.append((q_hbm, j, q_out.at[pl.ds(base, _K), pl.ds(0, d)],
                          bi_hbm, j, bi_out.at[pl.ds(base, _K)]))
            tasks.append((p_hbm, j,
                          pself_out.at[pl.ds(base, _K), pl.ds(0, d)],
                          bu_hbm, NQ + j, bu_out.at[pl.ds(base, _K)]))
        for j in range(NNCH):
            base = wid * NEGW + j * _K
            tasks.append((q_hbm, 2 * NQ + j,
                          qneg_out.at[pl.ds(base, _K), pl.ds(0, d)],
                          bi_hbm, 2 * NQ + j, bineg_out.at[pl.ds(base, _K)]))

        def fire(t):
            tab, ji, _, btab, bji, _ = tasks[t]
            descs = [pltpu.async_copy(tab.at[idx_v.at[pl.ds(ji * _K, _K)]],
                                      rbufs[t % 2], rsems[t % 2])]
            if btab is not None:
                descs.append(pltpu.async_copy(
                    btab.at[idx_v.at[pl.ds(bji * _K, _K)]],
                    bbufs[t % 2], bsems[t % 2]))
            return descs

        bbufs = (bva, bvb)
        pend = fire(0)
        for t in range(len(tasks)):
            nxt = fire(t + 1) if t + 1 < len(tasks) else []
            for dsc in pend:
                dsc.wait()
            pend = nxt
            _, _, out, btab, _, bout = tasks[t]
            pltpu.sync_copy(rbufs[t % 2], out)
            if btab is not None:
                pltpu.sync_copy(bbufs[t % 2], bout)

        plsc.subcore_barrier()

        @pl.loop(0, NCH, step=_G)
        def _edges(j):
            ebase = wid * EW + j * _K
            pltpu.sync_copy(iu1.at[pl.ds(ebase, _G * _K)], iu_g)
            sdescs = [pltpu.async_copy(us1.at[pl.ds(ebase + k * _K, _K)],
                                       us_g.at[k], rsem_a) for k in range(_G)]
            for dsc in sdescs:
                dsc.wait()
            pltpu.async_copy(p_hbm.at[iu_g.at[pl.ds(0, _K)]], gbufs[0],
                             gsems[0])
            pltpu.async_copy(p_hbm.at[iu_g.at[pl.ds(_K, _K)]], gbufs[1],
                             gsems[1])
            for k in range(_G):
                cur = gbufs[k % 3]
                pltpu.make_async_copy(p_hbm.at[iu_g.at[pl.ds(k * _K, _K)]],
                                      cur, gsems[k % 3]).wait()
                if k + 2 < _G:
                    pltpu.async_copy(
                        p_hbm.at[iu_g.at[pl.ds((k + 2) * _K, _K)]],
                        gbufs[(k + 2) % 3], gsems[(k + 2) % 3])
                pltpu.sync_copy(cur, psum_sh.at[us_g.at[k]], add=True)

        plsc.subcore_barrier()

        pltpu.sync_copy(psum_sh.at[pl.ds(sid * RPT, RPT)],
                        psum_out.at[cid, pl.ds(sid * RPT, RPT), pl.ds(0, d)])

    return sc_kernel, NC, NW, NCH, NNCH, NQ


def _tc_score(psum, q, pself, qneg3, bu, bi, bineg2, B, d, n_negs, NC):
    BLK = 2048
    grid = (B // BLK,)

    def body(psum_ref, q_ref, pself_ref, qneg_ref, bu_ref, bi_ref, bineg_ref,
             r_ref, rneg_ref):
        p_sum = psum_ref[0, :, :d]
        for c in range(1, NC):
            p_sum = p_sum + psum_ref[c, :, :d]
        p_ctx = p_sum - pself_ref[:, :d]
        pq = jnp.sum(p_ctx * q_ref[:, :d], axis=1)
        r_ref[...] = bu_ref[...] + bi_ref[...] + pq
        pqn = jnp.sum(p_ctx[:, None, :] * qneg_ref[:, :, :d], axis=2)
        rneg_ref[...] = bu_ref[...][:, None] + bineg_ref[...] + pqn

    return pl.pallas_call(
        body,
        grid=grid,
        in_specs=[
            pl.BlockSpec((NC, BLK, 128), lambda i: (0, i, 0)),
            pl.BlockSpec((BLK, 128), lambda i: (i, 0)),
            pl.BlockSpec((BLK, 128), lambda i: (i, 0)),
            pl.BlockSpec((BLK, n_negs, 128), lambda i: (i, 0, 0)),
            pl.BlockSpec((BLK,), lambda i: (i,)),
            pl.BlockSpec((BLK,), lambda i: (i,)),
            pl.BlockSpec((BLK, n_negs), lambda i: (i, 0)),
        ],
        out_specs=[
            pl.BlockSpec((BLK,), lambda i: (i,)),
            pl.BlockSpec((BLK, n_negs), lambda i: (i, 0)),
        ],
        out_shape=[
            jax.ShapeDtypeStruct((B,), jnp.float32),
            jax.ShapeDtypeStruct((B, n_negs), jnp.float32),
        ],
    )(psum, q, pself, qneg3, bu, bi, bineg2)


def kernel(P_table, Q_table, b_u, b_i, I, U, I_neg, I_U, U_idx):
    B = I.shape[0]
    n_negs = I_neg.shape[1]
    E = I_U.shape[0]
    n_items, d = P_table.shape
    n_users = b_u.shape[0]

    sc_kernel, NC, NW, NCH, NNCH, NQ = _make_sc_gather(
        n_items, n_users, B, E, d, n_negs)

    iu1 = I_U.astype(jnp.int32)
    us1 = U_idx.astype(jnp.int32)
    i1 = I.astype(jnp.int32)
    u1 = U.astype(jnp.int32)
    ineg1 = I_neg.astype(jnp.int32).reshape(-1)
    zeros = jnp.zeros((_K, d), jnp.float32)

    psum, q, pself, qneg, bu, bi, bineg = sc_kernel(
        P_table, Q_table, b_u, b_i, iu1, us1, i1, u1, ineg1, zeros)

    qneg3 = qneg.reshape(B, n_negs, 128)
    bineg2 = bineg.reshape(B, n_negs)
    r, rneg = _tc_score(psum, q, pself, qneg3, bu, bi, bineg2, B, d, n_negs, NC)
    return (r, rneg)


# speedup vs baseline: 14.2825x; 1.0071x over previous
import functools

import jax
import jax.numpy as jnp
from jax import lax
from jax.experimental import pallas as pl
from jax.experimental.pallas import tpu as pltpu
from jax.experimental.pallas import tpu_sc as plsc

_K = 128
_G = 10


@functools.lru_cache(maxsize=None)
def _make_sc_gather(n_items, n_users, B, E, d, n_negs):
    info = plsc.get_sparse_core_info()
    NC, NS = info.num_cores, info.num_subcores
    NW = NC * NS
    EW = E // NW
    NCH = EW // _K
    BW = B // NW
    NQ = BW // _K
    NEG = B * n_negs
    NEGW = NEG // NW
    NNCH = NEGW // _K
    RPT = B // NS

    mesh = plsc.VectorSubcoreMesh(core_axis_name="c", subcore_axis_name="s")

    @functools.partial(
        pl.kernel,
        out_type=(
            jax.ShapeDtypeStruct((NC, B, 128), jnp.float32),
            jax.ShapeDtypeStruct((B, 128), jnp.float32),
            jax.ShapeDtypeStruct((B, 128), jnp.float32),
            jax.ShapeDtypeStruct((NEG, 128), jnp.float32),
            jax.ShapeDtypeStruct((B,), jnp.float32),
            jax.ShapeDtypeStruct((B,), jnp.float32),
            jax.ShapeDtypeStruct((NEG,), jnp.float32),
        ),
        mesh=mesh,
        compiler_params=pltpu.CompilerParams(use_tc_tiling_on_sc=False),
        scratch_types=[
            pltpu.VMEM((_G * _K,), jnp.int32),
            pltpu.VMEM((_G, _K), jnp.int32),
            pltpu.VMEM((5, _K, d), jnp.float32),
            pltpu.VMEM(((2 * (B // _K // NW) + (NEG // _K // NW)) * _K,),
                       jnp.int32),
            pltpu.VMEM((_K,), jnp.float32),
            pltpu.VMEM((_K,), jnp.float32),
            pltpu.VMEM_SHARED((B, d), jnp.float32),
            pltpu.SemaphoreType.DMA,
            pltpu.SemaphoreType.DMA,
            pltpu.SemaphoreType.DMA,
            pltpu.SemaphoreType.DMA,
            pltpu.SemaphoreType.DMA,
            pltpu.SemaphoreType.DMA,
            pltpu.SemaphoreType.DMA,
            pltpu.SemaphoreType.DMA,
            pltpu.SemaphoreType.DMA,
            pltpu.SemaphoreType.DMA,
            pltpu.SemaphoreType.DMA,
            pltpu.SemaphoreType.DMA,
            pltpu.SemaphoreType.DMA,
            pltpu.SemaphoreType.DMA,
        ],
    )
    def sc_kernel(p_hbm, q_hbm, bu_hbm, bi_hbm, iu1, us1, i1, u1, ineg1, z_hbm,
                  psum_out, q_out, pself_out, qneg_out, bu_out, bi_out, bineg_out,
                  iu_g, us_g, ring, idx_v, bva, bvb, psum_sh,
                  g0, g1, g2, g3, g4, s0, s1, s2, s3, s4,
                  rsem_a, rsem_b, bsem_a, bsem_b):
        gsems = (g0, g1, g2, g3, g4)
        ssems = (s0, s1, s2, s3, s4)
        gbufs = tuple(ring.at[i] for i in range(5))
        rows_a = gbufs[0]
        rows_b = gbufs[1]

        rsems = (rsem_a, rsem_b)
        bsems = (bsem_a, bsem_b)
        cid = lax.axis_index("c")
        sid = lax.axis_index("s")
        wid = cid * NS + sid
        rbufs = (rows_a, rows_b)

        pltpu.sync_copy(z_hbm, rows_a)
        for j in range(RPT // _K):
            pltpu.sync_copy(rows_a, psum_sh.at[pl.ds(sid * RPT + j * _K, _K)])

        pltpu.sync_copy(i1.at[pl.ds(wid * BW, BW)], idx_v.at[pl.ds(0, BW)])
        pltpu.sync_copy(u1.at[pl.ds(wid * BW, BW)], idx_v.at[pl.ds(BW, BW)])
        pltpu.sync_copy(ineg1.at[pl.ds(wid * NEGW, NEGW)],
                        idx_v.at[pl.ds(2 * BW, NEGW)])

        tasks = []
        for j in range(NQ):
            base = wid * BW + j * _K
            tasks.append((q_hbm, j, q_out.at[pl.ds(base, _K), pl.ds(0, d)],
                          bi_hbm, j, bi_out.at[pl.ds(base, _K)]))
            tasks.append((p_hbm, j,
                          pself_out.at[pl.ds(base, _K), pl.ds(0, d)],
                          bu_hbm, NQ + j, bu_out.at[pl.ds(base, _K)]))
        for j in range(NNCH):
            base = wid * NEGW + j * _K
            tasks.append((q_hbm, 2 * NQ + j,
                          qneg_out.at[pl.ds(base, _K), pl.ds(0, d)],
                          bi_hbm, 2 * NQ + j, bineg_out.at[pl.ds(base, _K)]))

        def fire(t):
            tab, ji, _, btab, bji, _ = tasks[t]
            descs = [pltpu.async_copy(tab.at[idx_v.at[pl.ds(ji * _K, _K)]],
                                      rbufs[t % 2], rsems[t % 2])]
            if btab is not None:
                descs.append(pltpu.async_copy(
                    btab.at[idx_v.at[pl.ds(bji * _K, _K)]],
                    bbufs[t % 2], bsems[t % 2]))
            return descs

        bbufs = (bva, bvb)
        pend = fire(0)
        for t in range(len(tasks)):
            nxt = fire(t + 1) if t + 1 < len(tasks) else []
            for dsc in pend:
                dsc.wait()
            pend = nxt
            _, _, out, btab, _, bout = tasks[t]
            pltpu.sync_copy(rbufs[t % 2], out)
            if btab is not None:
                pltpu.sync_copy(bbufs[t % 2], bout)

        plsc.subcore_barrier()

        @pl.loop(0, NCH, step=_G)
        def _edges(j):
            ebase = wid * EW + j * _K
            pltpu.sync_copy(iu1.at[pl.ds(ebase, _G * _K)], iu_g)
            sdescs = [pltpu.async_copy(us1.at[pl.ds(ebase + k * _K, _K)],
                                       us_g.at[k], rsem_a) for k in range(_G)]
            for dsc in sdescs:
                dsc.wait()
            _A = 3
            for k0 in range(_A):
                pltpu.async_copy(p_hbm.at[iu_g.at[pl.ds(k0 * _K, _K)]],
                                 gbufs[k0 % 5], gsems[k0 % 5])
            sd = {}
            for k in range(_G):
                cur = gbufs[k % 5]
                pltpu.make_async_copy(p_hbm.at[iu_g.at[pl.ds(k * _K, _K)]],
                                      cur, gsems[k % 5]).wait()
                sd[k] = pltpu.async_copy(cur, psum_sh.at[us_g.at[k]],
                                         ssems[k % 5], add=True)
                if k + _A < _G:
                    kn = k + _A
                    if kn - 5 >= 0:
                        sd.pop(kn - 5).wait()
                    pltpu.async_copy(p_hbm.at[iu_g.at[pl.ds(kn * _K, _K)]],
                                     gbufs[kn % 5], gsems[kn % 5])
            for dsc in sd.values():
                dsc.wait()

        plsc.subcore_barrier()

        pltpu.sync_copy(psum_sh.at[pl.ds(sid * RPT, RPT)],
                        psum_out.at[cid, pl.ds(sid * RPT, RPT), pl.ds(0, d)])

    return sc_kernel, NC, NW, NCH, NNCH, NQ


def _tc_score(psum, q, pself, qneg3, bu, bi, bineg2, B, d, n_negs, NC):
    BLK = 2048
    grid = (B // BLK,)

    def body(psum_ref, q_ref, pself_ref, qneg_ref, bu_ref, bi_ref, bineg_ref,
             r_ref, rneg_ref):
        p_sum = psum_ref[0, :, :d]
        for c in range(1, NC):
            p_sum = p_sum + psum_ref[c, :, :d]
        p_ctx = p_sum - pself_ref[:, :d]
        pq = jnp.sum(p_ctx * q_ref[:, :d], axis=1)
        r_ref[...] = bu_ref[...] + bi_ref[...] + pq
        pqn = jnp.sum(p_ctx[:, None, :] * qneg_ref[:, :, :d], axis=2)
        rneg_ref[...] = bu_ref[...][:, None] + bineg_ref[...] + pqn

    return pl.pallas_call(
        body,
        grid=grid,
        in_specs=[
            pl.BlockSpec((NC, BLK, 128), lambda i: (0, i, 0)),
            pl.BlockSpec((BLK, 128), lambda i: (i, 0)),
            pl.BlockSpec((BLK, 128), lambda i: (i, 0)),
            pl.BlockSpec((BLK, n_negs, 128), lambda i: (i, 0, 0)),
            pl.BlockSpec((BLK,), lambda i: (i,)),
            pl.BlockSpec((BLK,), lambda i: (i,)),
            pl.BlockSpec((BLK, n_negs), lambda i: (i, 0)),
        ],
        out_specs=[
            pl.BlockSpec((BLK,), lambda i: (i,)),
            pl.BlockSpec((BLK, n_negs), lambda i: (i, 0)),
        ],
        out_shape=[
            jax.ShapeDtypeStruct((B,), jnp.float32),
            jax.ShapeDtypeStruct((B, n_negs), jnp.float32),
        ],
    )(psum, q, pself, qneg3, bu, bi, bineg2)


def kernel(P_table, Q_table, b_u, b_i, I, U, I_neg, I_U, U_idx):
    B = I.shape[0]
    n_negs = I_neg.shape[1]
    E = I_U.shape[0]
    n_items, d = P_table.shape
    n_users = b_u.shape[0]

    sc_kernel, NC, NW, NCH, NNCH, NQ = _make_sc_gather(
        n_items, n_users, B, E, d, n_negs)

    iu1 = I_U.astype(jnp.int32)
    us1 = U_idx.astype(jnp.int32)
    i1 = I.astype(jnp.int32)
    u1 = U.astype(jnp.int32)
    ineg1 = I_neg.astype(jnp.int32).reshape(-1)
    zeros = jnp.zeros((_K, d), jnp.float32)

    psum, q, pself, qneg, bu, bi, bineg = sc_kernel(
        P_table, Q_table, b_u, b_i, iu1, us1, i1, u1, ineg1, zeros)

    qneg3 = qneg.reshape(B, n_negs, 128)
    bineg2 = bineg.reshape(B, n_negs)
    r, rneg = _tc_score(psum, q, pself, qneg3, bu, bi, bineg2, B, d, n_negs, NC)
    return (r, rneg)
